# baseline (speedup 1.0000x reference)
import functools

import jax
import jax.numpy as jnp
from jax.experimental import pallas as pl
from jax.experimental.pallas import tpu as pltpu
from jax.experimental.pallas import tpu_sc as plsc

N = 10000
E = 320000
D = 128

NC = 2
NS = 16
L = 16

NPAD = 10112
ROWS_PER_SUB = NPAD // NS
K = 128
EPS = E // (NC * NS)
EPSP = 10240
GC = 8
NITER = EPSP // K
NG = NITER // GC


def _sc_aggregate(h, srcp, dstp, zeros2d, zeros1d, ones1d, with_deg):
  out_type = [jax.ShapeDtypeStruct((NC * NPAD, D), jnp.float32)]
  if with_deg:
    out_type.append(jax.ShapeDtypeStruct((NC * NPAD,), jnp.float32))

  scratch = [
      pltpu.VMEM((2 * GC * K,), jnp.int32),
      pltpu.VMEM((2, GC, K), jnp.int32),
      pltpu.VMEM((2, K, D), jnp.float32),
      pltpu.VMEM((K,), jnp.float32),
      pltpu.VMEM_SHARED((NPAD, D), jnp.float32),
      pltpu.VMEM_SHARED((NPAD,), jnp.float32),
      pltpu.SemaphoreType.DMA,
      pltpu.SemaphoreType.DMA,
      pltpu.SemaphoreType.DMA,
      pltpu.SemaphoreType.DMA,
      pltpu.SemaphoreType.DMA,
      pltpu.SemaphoreType.DMA,
  ]

  mesh = plsc.VectorSubcoreMesh(core_axis_name="c", subcore_axis_name="s")

  def body(h_hbm, src_hbm, dst_hbm, z2_hbm, z1_hbm, ones_hbm, *rest):
    if with_deg:
      (acc_out, deg_out, src_v, dst_v, rows_v, ones_v, acc_sh, deg_sh,
       sem_g0, sem_g1, sem_s0, sem_s1, sem_i, sem_d) = rest
    else:
      (acc_out, src_v, dst_v, rows_v, ones_v, acc_sh, deg_sh,
       sem_g0, sem_g1, sem_s0, sem_s1, sem_i, sem_d) = rest
      deg_out = None
    c = jax.lax.axis_index("c")
    s = jax.lax.axis_index("s")
    w = c * NS + s
    rbase = s * ROWS_PER_SUB
    gsems = (sem_g0, sem_g1)
    ssems = (sem_s0, sem_s1)

    pltpu.sync_copy(z2_hbm, acc_sh.at[pl.ds(rbase, ROWS_PER_SUB)])
    pltpu.sync_copy(src_hbm.at[w, pl.ds(0, GC * K)],
                    src_v.at[pl.ds(0, GC * K)])
    pltpu.sync_copy(dst_hbm.at[w, 0], dst_v.at[0])
    if with_deg:
      @pl.when(s == 0)
      def _():
        pltpu.sync_copy(z1_hbm, deg_sh)
      pltpu.sync_copy(ones_hbm, ones_v)
    plsc.subcore_barrier()

    def src_slice(half, i):
      off = pl.multiple_of(half * (GC * K) + i * K, 8)
      return src_v.at[pl.ds(off, K)]

    def issue_gather(half, i, b):
      pltpu.async_copy(h_hbm.at[src_slice(half, i)], rows_v.at[b], gsems[b])

    def wait_gather(half, i, b):
      pltpu.make_async_copy(h_hbm.at[src_slice(half, i)], rows_v.at[b],
                            gsems[b]).wait()

    def wait_deg():
      pltpu.make_async_copy(ones_v, deg_sh.at[dst_v.at[0, 0]], sem_d).wait()

    def issue_scatter(half, i, b):
      pltpu.async_copy(rows_v.at[b], acc_sh.at[dst_v.at[half, i]], ssems[b],
                       add=True)

    def wait_scatter(b):
      pltpu.make_async_copy(rows_v.at[b], acc_sh.at[dst_v.at[0, 0]],
                            ssems[b]).wait()

    issue_gather(0, 0, 0)

    def group(g, carry):
      half = jax.lax.rem(g, 2)
      nhalf = 1 - half

      @pl.when(g < NG - 1)
      def _():
        soff = pl.multiple_of((g + 1) * GC * K, 8)
        pltpu.async_copy(src_hbm.at[w, pl.ds(soff, GC * K)],
                         src_v.at[pl.ds(nhalf * (GC * K), GC * K)], sem_i)
        pltpu.async_copy(dst_hbm.at[w, g + 1], dst_v.at[nhalf], sem_i)

      for i in range(GC):
        b = i % 2
        nb = 1 - b
        if i == 0:
          @pl.when(g > 0)
          def _():
            wait_scatter(nb)
        else:
          wait_scatter(nb)
        if i < GC - 1:
          issue_gather(half, i + 1, nb)
        else:
          @pl.when(g < NG - 1)
          def _():
            pltpu.make_async_copy(
                src_hbm.at[w, pl.ds(pl.multiple_of((g + 1) * GC * K, 8),
                                    GC * K)],
                src_v.at[pl.ds(nhalf * (GC * K), GC * K)], sem_i).wait()
            pltpu.make_async_copy(dst_hbm.at[w, g + 1], dst_v.at[nhalf],
                                  sem_i).wait()
            issue_gather(nhalf, 0, nb)
        wait_gather(half, i, b)
        issue_scatter(half, i, b)
        if with_deg:
          if i == 0:
            @pl.when(g > 0)
            def _():
              wait_deg()
          else:
            wait_deg()
          pltpu.async_copy(ones_v, deg_sh.at[dst_v.at[half, i]], sem_d,
                           add=True)
      return carry

    jax.lax.fori_loop(0, NG, group, 0)
    wait_scatter(1)
    if with_deg:
      wait_deg()
    plsc.subcore_barrier()

    obase = c * NPAD + rbase
    pltpu.sync_copy(acc_sh.at[pl.ds(rbase, ROWS_PER_SUB)],
                    acc_out.at[pl.ds(obase, ROWS_PER_SUB)])
    if with_deg:
      @pl.when(s == 0)
      def _():
        pltpu.sync_copy(deg_sh, deg_out.at[pl.ds(c * NPAD, NPAD)])

  fn = pl.kernel(body, out_type=out_type, mesh=mesh, scratch_types=scratch,
                 name="sc_gather_scatter_add")
  res = fn(h, srcp, dstp, zeros2d, zeros1d, ones1d)
  if with_deg:
    return res[0], res[1]
  return res[0], None


def _tc_self(x, Ws):
  def body(x_ref, Ws_ref, o_ref):
    o_ref[...] = jnp.dot(x_ref[...], Ws_ref[...],
                         preferred_element_type=jnp.float32)

  RB = 2000
  return pl.pallas_call(
      body,
      grid=(N // RB,),
      in_specs=[
          pl.BlockSpec((RB, D), lambda i: (i, 0)),
          pl.BlockSpec((D, D), lambda i: (0, 0)),
      ],
      out_specs=pl.BlockSpec((RB, D), lambda i: (i, 0)),
      out_shape=jax.ShapeDtypeStruct((N, D), jnp.float32),
  )(x, Ws)


def _tc_layer1(xs, acc, degp, Wn, b):
  def body(xs_ref, acc_ref, degp_ref, Wn_ref, b_ref, h_ref):
    deg = degp_ref[0] + degp_ref[1]
    invdeg = 1.0 / jnp.maximum(deg, 1.0)
    agg = (acc_ref[0] + acc_ref[1]) * invdeg
    z = (xs_ref[...]
         + jnp.dot(agg, Wn_ref[...], preferred_element_type=jnp.float32)
         + b_ref[...])
    h_ref[...] = jnp.maximum(z, 0.0)

  RB = 2000
  return pl.pallas_call(
      body,
      grid=(N // RB,),
      in_specs=[
          pl.BlockSpec((RB, D), lambda i: (i, 0)),
          pl.BlockSpec((NC, RB, D), lambda i: (0, i, 0)),
          pl.BlockSpec((NC, RB, 1), lambda i: (0, i, 0)),
          pl.BlockSpec((D, D), lambda i: (0, 0)),
          pl.BlockSpec((1, D), lambda i: (0, 0)),
      ],
      out_specs=pl.BlockSpec((RB, D), lambda i: (i, 0)),
      out_shape=jax.ShapeDtypeStruct((N, D), jnp.float32),
  )(xs, acc.reshape(NC, NPAD, D), degp.reshape(NC, NPAD, 1), Wn,
    b.reshape(1, -1))


def _tc_layer2_head(h1, acc, degp, Ws, Wn, b, Wc1, bc1, Wc2, bc2):
  def body(h1_ref, acc_ref, degp_ref, Ws_ref, Wn_ref, b_ref, Wc1_ref,
           bc1_ref, Wc2_ref, bc2_ref, out_ref):
    deg = degp_ref[0] + degp_ref[1]
    invdeg = 1.0 / jnp.maximum(deg, 1.0)
    agg = (acc_ref[0] + acc_ref[1]) * invdeg
    h1v = h1_ref[...]
    z = (jnp.dot(h1v, Ws_ref[...], preferred_element_type=jnp.float32)
         + jnp.dot(agg, Wn_ref[...], preferred_element_type=jnp.float32)
         + b_ref[...])
    h2 = jnp.maximum(z, 0.0)
    hc = (jnp.dot(h1v, Wc1_ref[:D, :], preferred_element_type=jnp.float32)
          + jnp.dot(h2, Wc1_ref[D:, :], preferred_element_type=jnp.float32)
          + bc1_ref[...])
    hc = jnp.maximum(hc, 0.0)
    out_ref[...] = (jnp.dot(hc, Wc2_ref[...],
                            preferred_element_type=jnp.float32)
                    + bc2_ref[...])

  C = bc2.shape[0]
  CH = Wc2.shape[0]
  RB = 2000
  return pl.pallas_call(
      body,
      grid=(N // RB,),
      in_specs=[
          pl.BlockSpec((RB, D), lambda i: (i, 0)),
          pl.BlockSpec((NC, RB, D), lambda i: (0, i, 0)),
          pl.BlockSpec((NC, RB, 1), lambda i: (0, i, 0)),
          pl.BlockSpec((D, D), lambda i: (0, 0)),
          pl.BlockSpec((D, D), lambda i: (0, 0)),
          pl.BlockSpec((1, D), lambda i: (0, 0)),
          pl.BlockSpec((2 * D, CH), lambda i: (0, 0)),
          pl.BlockSpec((1, CH), lambda i: (0, 0)),
          pl.BlockSpec((CH, C), lambda i: (0, 0)),
          pl.BlockSpec((1, C), lambda i: (0, 0)),
      ],
      out_specs=pl.BlockSpec((RB, C), lambda i: (i, 0)),
      out_shape=jax.ShapeDtypeStruct((N, C), jnp.float32),
  )(h1, acc.reshape(NC, NPAD, D), degp.reshape(NC, NPAD, 1), Ws, Wn,
    b.reshape(1, -1),
    Wc1, bc1.reshape(1, -1), Wc2, bc2.reshape(1, -1))


def kernel(x, edge_index, Ws1, Wn1, b1, Ws2, Wn2, b2, Wc1, bc1, Wc2, bc2):
  NW = NC * NS
  npad_e = NW * (EPSP - EPS)
  pad_src = jnp.arange(npad_e, dtype=jnp.int32) % N
  pad_dst = (N + jnp.arange(npad_e, dtype=jnp.int32) % (NPAD - N)
             ).astype(jnp.int32)
  src = jnp.concatenate([edge_index[0], pad_src]).reshape(NW, EPSP)
  dst = jnp.concatenate([edge_index[1], pad_dst]).reshape(NW, NG, GC, K)
  zeros2d = jnp.zeros((ROWS_PER_SUB, D), jnp.float32)
  zeros1d = jnp.zeros((NPAD,), jnp.float32)
  ones1d = jnp.ones((K,), jnp.float32)

  xs = _tc_self(x, Ws1)
  acc1, degp = _sc_aggregate(x, src, dst, zeros2d, zeros1d, ones1d,
                             with_deg=True)
  h1 = _tc_layer1(xs, acc1, degp, Wn1, b1)
  acc2, _ = _sc_aggregate(h1, src, dst, zeros2d, zeros1d, ones1d,
                          with_deg=False)
  return _tc_layer2_head(h1, acc2, degp, Ws2, Wn2, b2, Wc1, bc1, Wc2, bc2)


# speedup vs baseline: 1.0135x; 1.0135x over previous
import functools

import jax
import jax.numpy as jnp
from jax.experimental import pallas as pl
from jax.experimental.pallas import tpu as pltpu
from jax.experimental.pallas import tpu_sc as plsc

N = 10000
E = 320000
D = 128

NC = 2
NS = 16
L = 16

NPAD = 10112
ROWS_PER_SUB = NPAD // NS
K = 128
EPS = E // (NC * NS)
EPSP = 10240
GC = 8
NITER = EPSP // K
NG = NITER // GC


def _sc_aggregate(h, srcp, dstp, zeros2d, zeros1d, ones1d, with_deg):
  out_type = [jax.ShapeDtypeStruct((NC * NPAD, D), jnp.float32)]
  if with_deg:
    out_type.append(jax.ShapeDtypeStruct((NC * NPAD,), jnp.float32))

  scratch = [
      pltpu.VMEM((2 * GC * K,), jnp.int32),
      pltpu.VMEM((2, GC, K), jnp.int32),
      pltpu.VMEM((2, K, D), jnp.float32),
      pltpu.VMEM((K,), jnp.float32),
      pltpu.VMEM_SHARED((NPAD, D), jnp.float32),
      pltpu.VMEM_SHARED((NPAD,), jnp.float32),
      pltpu.SemaphoreType.DMA,
      pltpu.SemaphoreType.DMA,
      pltpu.SemaphoreType.DMA,
      pltpu.SemaphoreType.DMA,
      pltpu.SemaphoreType.DMA,
      pltpu.SemaphoreType.DMA,
  ]

  mesh = plsc.VectorSubcoreMesh(core_axis_name="c", subcore_axis_name="s")

  def body(h_hbm, src_hbm, dst_hbm, z2_hbm, z1_hbm, ones_hbm, *rest):
    if with_deg:
      (acc_out, deg_out, src_v, dst_v, rows_v, ones_v, acc_sh, deg_sh,
       sem_g0, sem_g1, sem_s0, sem_s1, sem_i, sem_d) = rest
    else:
      (acc_out, src_v, dst_v, rows_v, ones_v, acc_sh, deg_sh,
       sem_g0, sem_g1, sem_s0, sem_s1, sem_i, sem_d) = rest
      deg_out = None
    c = jax.lax.axis_index("c")
    s = jax.lax.axis_index("s")
    w = c * NS + s
    rbase = s * ROWS_PER_SUB
    gsems = (sem_g0, sem_g1)
    ssems = (sem_s0, sem_s1)

    pltpu.sync_copy(z2_hbm, acc_sh.at[pl.ds(rbase, ROWS_PER_SUB)])
    pltpu.sync_copy(src_hbm.at[w, pl.ds(0, GC * K)],
                    src_v.at[pl.ds(0, GC * K)])
    pltpu.sync_copy(dst_hbm.at[w, 0], dst_v.at[0])
    if with_deg:
      @pl.when(s == 0)
      def _():
        pltpu.sync_copy(z1_hbm, deg_sh)
      pltpu.sync_copy(ones_hbm, ones_v)
    plsc.subcore_barrier()

    def src_slice(half, i):
      off = pl.multiple_of(half * (GC * K) + i * K, 8)
      return src_v.at[pl.ds(off, K)]

    def issue_gather(half, i, b):
      pltpu.async_copy(h_hbm.at[src_slice(half, i)], rows_v.at[b], gsems[b])

    def wait_gather(half, i, b):
      pltpu.make_async_copy(h_hbm.at[src_slice(half, i)], rows_v.at[b],
                            gsems[b]).wait()

    def wait_deg():
      pltpu.make_async_copy(ones_v, deg_sh.at[dst_v.at[0, 0]], sem_d).wait()

    def issue_scatter(half, i, b):
      pltpu.async_copy(rows_v.at[b], acc_sh.at[dst_v.at[half, i]], ssems[b],
                       add=True)

    def wait_scatter(b):
      pltpu.make_async_copy(rows_v.at[b], acc_sh.at[dst_v.at[0, 0]],
                            ssems[b]).wait()

    issue_gather(0, 0, 0)

    def group(g, carry):
      half = jax.lax.rem(g, 2)
      nhalf = 1 - half

      @pl.when(g < NG - 1)
      def _():
        soff = pl.multiple_of((g + 1) * GC * K, 8)
        pltpu.async_copy(src_hbm.at[w, pl.ds(soff, GC * K)],
                         src_v.at[pl.ds(nhalf * (GC * K), GC * K)], sem_i)
        pltpu.async_copy(dst_hbm.at[w, g + 1], dst_v.at[nhalf], sem_i)

      for i in range(GC):
        b = i % 2
        nb = 1 - b
        if i == 0:
          @pl.when(g > 0)
          def _():
            wait_scatter(nb)
        else:
          wait_scatter(nb)
        if i < GC - 1:
          issue_gather(half, i + 1, nb)
        else:
          @pl.when(g < NG - 1)
          def _():
            pltpu.make_async_copy(
                src_hbm.at[w, pl.ds(pl.multiple_of((g + 1) * GC * K, 8),
                                    GC * K)],
                src_v.at[pl.ds(nhalf * (GC * K), GC * K)], sem_i).wait()
            pltpu.make_async_copy(dst_hbm.at[w, g + 1], dst_v.at[nhalf],
                                  sem_i).wait()
            issue_gather(nhalf, 0, nb)
        wait_gather(half, i, b)
        issue_scatter(half, i, b)
        if with_deg:
          if i == 0:
            @pl.when(g > 0)
            def _():
              wait_deg()
          else:
            wait_deg()
          pltpu.async_copy(ones_v, deg_sh.at[dst_v.at[half, i]], sem_d,
                           add=True)
      return carry

    jax.lax.fori_loop(0, NG, group, 0)
    wait_scatter(1)
    if with_deg:
      wait_deg()
    plsc.subcore_barrier()

    obase = c * NPAD + rbase
    pltpu.sync_copy(acc_sh.at[pl.ds(rbase, ROWS_PER_SUB)],
                    acc_out.at[pl.ds(obase, ROWS_PER_SUB)])
    if with_deg:
      @pl.when(s == 0)
      def _():
        pltpu.sync_copy(deg_sh, deg_out.at[pl.ds(c * NPAD, NPAD)])

  fn = pl.kernel(body, out_type=out_type, mesh=mesh, scratch_types=scratch,
                 name="sc_gather_scatter_add")
  res = fn(h, srcp, dstp, zeros2d, zeros1d, ones1d)
  if with_deg:
    return res[0], res[1]
  return res[0], None


def _tc_layer1(x, acc, degp, Ws, Wn, b):
  def body(x_ref, acc_ref, degp_ref, Ws_ref, Wn_ref, b_ref, h_ref):
    deg = degp_ref[0] + degp_ref[1]
    invdeg = 1.0 / jnp.maximum(deg, 1.0)
    agg = (acc_ref[0] + acc_ref[1]) * invdeg
    z = (jnp.dot(x_ref[...], Ws_ref[...], preferred_element_type=jnp.float32)
         + jnp.dot(agg, Wn_ref[...], preferred_element_type=jnp.float32)
         + b_ref[...])
    h_ref[...] = jnp.maximum(z, 0.0)

  RB = 2000
  return pl.pallas_call(
      body,
      grid=(N // RB,),
      in_specs=[
          pl.BlockSpec((RB, D), lambda i: (i, 0)),
          pl.BlockSpec((NC, RB, D), lambda i: (0, i, 0)),
          pl.BlockSpec((NC, RB, 1), lambda i: (0, i, 0)),
          pl.BlockSpec((D, D), lambda i: (0, 0)),
          pl.BlockSpec((D, D), lambda i: (0, 0)),
          pl.BlockSpec((1, D), lambda i: (0, 0)),
      ],
      out_specs=pl.BlockSpec((RB, D), lambda i: (i, 0)),
      out_shape=jax.ShapeDtypeStruct((N, D), jnp.float32),
  )(x, acc.reshape(NC, NPAD, D), degp.reshape(NC, NPAD, 1), Ws, Wn,
    b.reshape(1, -1))


def _tc_layer2_head(h1, acc, degp, Ws, Wn, b, Wc1, bc1, Wc2, bc2):
  def body(h1_ref, acc_ref, degp_ref, Ws_ref, Wn_ref, b_ref, Wc1_ref,
           bc1_ref, Wc2_ref, bc2_ref, out_ref):
    deg = degp_ref[0] + degp_ref[1]
    invdeg = 1.0 / jnp.maximum(deg, 1.0)
    agg = (acc_ref[0] + acc_ref[1]) * invdeg
    h1v = h1_ref[...]
    z = (jnp.dot(h1v, Ws_ref[...], preferred_element_type=jnp.float32)
         + jnp.dot(agg, Wn_ref[...], preferred_element_type=jnp.float32)
         + b_ref[...])
    h2 = jnp.maximum(z, 0.0)
    hc = (jnp.dot(h1v, Wc1_ref[:D, :], preferred_element_type=jnp.float32)
          + jnp.dot(h2, Wc1_ref[D:, :], preferred_element_type=jnp.float32)
          + bc1_ref[...])
    hc = jnp.maximum(hc, 0.0)
    out_ref[...] = (jnp.dot(hc, Wc2_ref[...],
                            preferred_element_type=jnp.float32)
                    + bc2_ref[...])

  C = bc2.shape[0]
  CH = Wc2.shape[0]
  RB = 2000
  return pl.pallas_call(
      body,
      grid=(N // RB,),
      in_specs=[
          pl.BlockSpec((RB, D), lambda i: (i, 0)),
          pl.BlockSpec((NC, RB, D), lambda i: (0, i, 0)),
          pl.BlockSpec((NC, RB, 1), lambda i: (0, i, 0)),
          pl.BlockSpec((D, D), lambda i: (0, 0)),
          pl.BlockSpec((D, D), lambda i: (0, 0)),
          pl.BlockSpec((1, D), lambda i: (0, 0)),
          pl.BlockSpec((2 * D, CH), lambda i: (0, 0)),
          pl.BlockSpec((1, CH), lambda i: (0, 0)),
          pl.BlockSpec((CH, C), lambda i: (0, 0)),
          pl.BlockSpec((1, C), lambda i: (0, 0)),
      ],
      out_specs=pl.BlockSpec((RB, C), lambda i: (i, 0)),
      out_shape=jax.ShapeDtypeStruct((N, C), jnp.float32),
  )(h1, acc.reshape(NC, NPAD, D), degp.reshape(NC, NPAD, 1), Ws, Wn,
    b.reshape(1, -1),
    Wc1, bc1.reshape(1, -1), Wc2, bc2.reshape(1, -1))


def kernel(x, edge_index, Ws1, Wn1, b1, Ws2, Wn2, b2, Wc1, bc1, Wc2, bc2):
  NW = NC * NS
  npad_e = NW * (EPSP - EPS)
  pad_src = jnp.arange(npad_e, dtype=jnp.int32) % N
  pad_dst = (N + jnp.arange(npad_e, dtype=jnp.int32) % (NPAD - N)
             ).astype(jnp.int32)
  src = jnp.concatenate([edge_index[0], pad_src]).reshape(NW, EPSP)
  dst = jnp.concatenate([edge_index[1], pad_dst]).reshape(NW, NG, GC, K)
  zeros2d = jnp.zeros((ROWS_PER_SUB, D), jnp.float32)
  zeros1d = jnp.zeros((NPAD,), jnp.float32)
  ones1d = jnp.ones((K,), jnp.float32)

  acc1, degp = _sc_aggregate(x, src, dst, zeros2d, zeros1d, ones1d,
                             with_deg=True)
  h1 = _tc_layer1(x, acc1, degp, Ws1, Wn1, b1)
  acc2, _ = _sc_aggregate(h1, src, dst, zeros2d, zeros1d, ones1d,
                          with_deg=False)
  return _tc_layer2_head(h1, acc2, degp, Ws2, Wn2, b2, Wc1, bc1, Wc2, bc2)
